# Initial kernel scaffold; baseline (speedup 1.0000x reference)
import jax
import jax.numpy as jnp
from jax.experimental import pallas as pl


def kernel(x, edge_index, W1_l, b1_l, W1_r, b1_r, W2_l, b2_l, W2_r, b2_r):
    raise NotImplementedError("write your pallas kernel here")


# speedup vs baseline: 12.5321x; 12.5321x over previous
import functools

import jax
import jax.numpy as jnp
from jax import lax
from jax.experimental import pallas as pl
from jax.experimental.pallas import tpu as pltpu
from jax.experimental.pallas import tpu_sc as plsc

_NC = 2
_NS = 16
_NW = _NC * _NS
_CH = 128
_F = 16


def _make_agg(n_pad, kpt, with_deg):
  rows_pt = n_pad // _NS
  mesh = plsc.VectorSubcoreMesh(core_axis_name="c", subcore_axis_name="s")
  out_type = [jax.ShapeDtypeStruct((_NC, n_pad, _F), jnp.float32)]
  scratch = [
      pltpu.VMEM((kpt, _CH), jnp.int32),
      pltpu.VMEM((kpt, _CH), jnp.int32),
      pltpu.VMEM((_CH, _F), jnp.float32),
      pltpu.VMEM((rows_pt, _F), jnp.float32),
      pltpu.VMEM_SHARED((n_pad, _F), jnp.float32),
      pltpu.SemaphoreType.DMA,
  ]
  if with_deg:
    out_type.append(jax.ShapeDtypeStruct((_NC, n_pad, _F), jnp.float32))
    scratch += [
        pltpu.VMEM((_CH, _F), jnp.float32),
        pltpu.VMEM_SHARED((n_pad, _F), jnp.float32),
    ]

  def body(src_hbm, dst_hbm, tab_hbm, *refs):
    if with_deg:
      (out_hbm, deg_hbm, src_v, dst_v, rows_v, tmp_v, acc_sh, sem,
       ones_v, deg_sh) = refs
    else:
      out_hbm, src_v, dst_v, rows_v, tmp_v, acc_sh, sem = refs
    c = lax.axis_index("c")
    s = lax.axis_index("s")
    wid = s * _NC + c
    base = s * rows_pt

    zero16 = jnp.zeros((_F,), jnp.float32)

    def zr(i, carry):
      tmp_v[i, :] = zero16
      return carry

    lax.fori_loop(0, rows_pt, zr, 0)
    pltpu.sync_copy(tmp_v, acc_sh.at[pl.ds(base, rows_pt)])
    if with_deg:
      pltpu.sync_copy(tmp_v, deg_sh.at[pl.ds(base, rows_pt)])
      one16 = jnp.ones((_F,), jnp.float32)

      def onr(i, carry):
        ones_v[i, :] = one16
        return carry

      lax.fori_loop(0, _CH, onr, 0)
    pltpu.sync_copy(src_hbm.at[wid], src_v)
    pltpu.sync_copy(dst_hbm.at[wid], dst_v)
    plsc.subcore_barrier()

    def chunk(j, carry):
      pltpu.async_copy(tab_hbm.at[src_v.at[j]], rows_v, sem).wait()
      pltpu.sync_copy(rows_v, acc_sh.at[dst_v.at[j]], add=True)
      if with_deg:
        pltpu.sync_copy(ones_v, deg_sh.at[dst_v.at[j]], add=True)
      return carry

    lax.fori_loop(0, kpt, chunk, 0)
    plsc.subcore_barrier()

    pltpu.sync_copy(acc_sh.at[pl.ds(base, rows_pt)], tmp_v)
    pltpu.sync_copy(tmp_v, out_hbm.at[c].at[pl.ds(base, rows_pt)])
    if with_deg:
      pltpu.sync_copy(deg_sh.at[pl.ds(base, rows_pt)], tmp_v)
      pltpu.sync_copy(tmp_v, deg_hbm.at[c].at[pl.ds(base, rows_pt)])

  return pl.kernel(
      body, out_type=out_type, mesh=mesh, scratch_types=scratch,
      compiler_params=pltpu.CompilerParams(use_tc_tiling_on_sc=False))


def _dense_in(x, w_l, w_r, b, blk):
  n, d = x.shape
  f = w_l.shape[1]

  def body(x_ref, wl_ref, wr_ref, b_ref, y_ref, s_ref):
    xb = x_ref[...]
    y_ref[...] = jnp.dot(xb, wl_ref[...], preferred_element_type=jnp.float32)
    s_ref[...] = (jnp.dot(xb, wr_ref[...], preferred_element_type=jnp.float32)
                  + b_ref[...])

  return pl.pallas_call(
      body,
      grid=(n // blk,),
      in_specs=[
          pl.BlockSpec((blk, d), lambda i: (i, 0)),
          pl.BlockSpec((d, f), lambda i: (0, 0)),
          pl.BlockSpec((d, f), lambda i: (0, 0)),
          pl.BlockSpec((1, f), lambda i: (0, 0)),
      ],
      out_specs=[
          pl.BlockSpec((blk, f), lambda i: (i, 0)),
          pl.BlockSpec((blk, f), lambda i: (i, 0)),
      ],
      out_shape=[
          jax.ShapeDtypeStruct((n, f), jnp.float32),
          jax.ShapeDtypeStruct((n, f), jnp.float32),
      ],
  )(x, w_l, w_r, b.reshape(1, f))


def _dense_mid(p1, dg, s1, w2_r, b2, blk):
  n, f = s1.shape
  n_pad = p1.shape[1]
  k = w2_r.shape[1]

  def body(p_ref, d_ref, s1_ref, w_ref, b_ref, h_ref, s2_ref):
    p = p_ref[0] + p_ref[1]
    deg = d_ref[0, :, 0:1] + d_ref[1, :, 0:1]
    h = jnp.maximum(p / jnp.maximum(deg, 1.0) + s1_ref[...], 0.0)
    h_ref[...] = h
    s2_ref[...] = (jnp.dot(h, w_ref[...], preferred_element_type=jnp.float32)
                   + b_ref[...])

  return pl.pallas_call(
      body,
      grid=(n // blk,),
      in_specs=[
          pl.BlockSpec((_NC, blk, _F), lambda i: (0, i, 0)),
          pl.BlockSpec((_NC, blk, _F), lambda i: (0, i, 0)),
          pl.BlockSpec((blk, f), lambda i: (i, 0)),
          pl.BlockSpec((f, k), lambda i: (0, 0)),
          pl.BlockSpec((1, k), lambda i: (0, 0)),
      ],
      out_specs=[
          pl.BlockSpec((blk, f), lambda i: (i, 0)),
          pl.BlockSpec((blk, k), lambda i: (i, 0)),
      ],
      out_shape=[
          jax.ShapeDtypeStruct((n, f), jnp.float32),
          jax.ShapeDtypeStruct((n, k), jnp.float32),
      ],
  )(p1, dg, s1, w2_r, b2.reshape(1, k))


def _dense_out(p2, dg, s2, w2_l, blk):
  n, k = s2.shape
  f = w2_l.shape[0]

  def body(p_ref, d_ref, s2_ref, w_ref, o_ref):
    p = p_ref[0] + p_ref[1]
    deg = d_ref[0, :, 0:1] + d_ref[1, :, 0:1]
    agg = p / jnp.maximum(deg, 1.0)
    lg = (jnp.dot(agg, w_ref[...], preferred_element_type=jnp.float32)
          + s2_ref[...])
    m = jnp.max(lg, axis=1, keepdims=True)
    lse = jnp.log(jnp.sum(jnp.exp(lg - m), axis=1, keepdims=True))
    o_ref[...] = lg - m - lse

  return pl.pallas_call(
      body,
      grid=(n // blk,),
      in_specs=[
          pl.BlockSpec((_NC, blk, _F), lambda i: (0, i, 0)),
          pl.BlockSpec((_NC, blk, _F), lambda i: (0, i, 0)),
          pl.BlockSpec((blk, k), lambda i: (i, 0)),
          pl.BlockSpec((f, k), lambda i: (0, 0)),
      ],
      out_specs=pl.BlockSpec((blk, k), lambda i: (i, 0)),
      out_shape=jax.ShapeDtypeStruct((n, k), jnp.float32),
  )(p2, dg, s2, w2_l)


def kernel(x, edge_index, W1_l, b1_l, W1_r, b1_r, W2_l, b2_l, W2_r, b2_r):
  n = x.shape[0]
  e = edge_index.shape[1]
  blk = 1000

  kpt = -(-e // (_NW * _CH))
  e_pad = _NW * _CH * kpt
  n_pad = -(-(n + 1) // (_NS * 8)) * (_NS * 8)

  src = edge_index[0].astype(jnp.int32)
  dst = edge_index[1].astype(jnp.int32)
  src_r = jnp.concatenate(
      [src, jnp.zeros((e_pad - e,), jnp.int32)]).reshape(_NW, kpt, _CH)
  dst_r = jnp.concatenate(
      [dst, jnp.full((e_pad - e,), n, jnp.int32)]).reshape(_NW, kpt, _CH)

  y1, s1 = _dense_in(x, W1_l, W1_r, b1_l + b1_r, blk)
  p1, dg = _make_agg(n_pad, kpt, True)(src_r, dst_r, y1)
  h, s2 = _dense_mid(p1, dg, s1, W2_r, b2_l + b2_r, blk)
  (p2,) = _make_agg(n_pad, kpt, False)(src_r, dst_r, h)
  return _dense_out(p2, dg, s2, W2_l, blk)


# speedup vs baseline: 18.7385x; 1.4952x over previous
import functools

import jax
import jax.numpy as jnp
from jax import lax
from jax.experimental import pallas as pl
from jax.experimental.pallas import tpu as pltpu
from jax.experimental.pallas import tpu_sc as plsc

_NC = 2
_NS = 16
_NW = _NC * _NS
_CH = 128
_F = 16


def _make_agg(n_pad, kpt, with_deg):
  rows_pt = n_pad // _NS
  mesh = plsc.VectorSubcoreMesh(core_axis_name="c", subcore_axis_name="s")
  out_type = [jax.ShapeDtypeStruct((_NC, n_pad, _F), jnp.float32)]
  scratch = [
      pltpu.VMEM((kpt, _CH), jnp.int32),
      pltpu.VMEM((kpt, _CH), jnp.int32),
      pltpu.VMEM((_CH, _F), jnp.float32),
      pltpu.VMEM((rows_pt, _F), jnp.float32),
      pltpu.VMEM_SHARED((n_pad, _F), jnp.float32),
      pltpu.VMEM_SHARED((n_pad, _F), jnp.float32),
      pltpu.SemaphoreType.DMA,
  ]
  if with_deg:
    out_type.append(jax.ShapeDtypeStruct((_NC, n_pad, _F), jnp.float32))
    scratch += [
        pltpu.VMEM((_CH, _F), jnp.float32),
        pltpu.VMEM_SHARED((n_pad, _F), jnp.float32),
    ]

  def body(src_hbm, dst_hbm, tab_hbm, *refs):
    if with_deg:
      (out_hbm, deg_hbm, src_v, dst_v, rows_v, tmp_v, acc_sh, tab_sh, sem,
       ones_v, deg_sh) = refs
    else:
      out_hbm, src_v, dst_v, rows_v, tmp_v, acc_sh, tab_sh, sem = refs
    c = lax.axis_index("c")
    s = lax.axis_index("s")
    wid = s * _NC + c
    base = s * rows_pt

    pltpu.sync_copy(tab_hbm.at[pl.ds(base, rows_pt)], tmp_v)
    pltpu.sync_copy(tmp_v, tab_sh.at[pl.ds(base, rows_pt)])

    zero16 = jnp.zeros((_F,), jnp.float32)

    def zr(i, carry):
      tmp_v[i, :] = zero16
      return carry

    lax.fori_loop(0, rows_pt, zr, 0)
    pltpu.sync_copy(tmp_v, acc_sh.at[pl.ds(base, rows_pt)])
    if with_deg:
      pltpu.sync_copy(tmp_v, deg_sh.at[pl.ds(base, rows_pt)])
      one16 = jnp.ones((_F,), jnp.float32)

      def onr(i, carry):
        ones_v[i, :] = one16
        return carry

      lax.fori_loop(0, _CH, onr, 0)
    pltpu.sync_copy(src_hbm.at[wid], src_v)
    pltpu.sync_copy(dst_hbm.at[wid], dst_v)
    plsc.subcore_barrier()

    def chunk(j, carry):
      pltpu.async_copy(tab_sh.at[src_v.at[j]], rows_v, sem).wait()
      pltpu.sync_copy(rows_v, acc_sh.at[dst_v.at[j]], add=True)
      if with_deg:
        pltpu.sync_copy(ones_v, deg_sh.at[dst_v.at[j]], add=True)
      return carry

    lax.fori_loop(0, kpt, chunk, 0)
    plsc.subcore_barrier()

    pltpu.sync_copy(acc_sh.at[pl.ds(base, rows_pt)], tmp_v)
    pltpu.sync_copy(tmp_v, out_hbm.at[c].at[pl.ds(base, rows_pt)])
    if with_deg:
      pltpu.sync_copy(deg_sh.at[pl.ds(base, rows_pt)], tmp_v)
      pltpu.sync_copy(tmp_v, deg_hbm.at[c].at[pl.ds(base, rows_pt)])

  return pl.kernel(
      body, out_type=out_type, mesh=mesh, scratch_types=scratch,
      compiler_params=pltpu.CompilerParams(use_tc_tiling_on_sc=False))


def _dense_in(x, w_l, w_r, b, blk):
  n, d = x.shape
  f = w_l.shape[1]

  def body(x_ref, wl_ref, wr_ref, b_ref, y_ref, s_ref):
    xb = x_ref[...]
    y_ref[...] = jnp.dot(xb, wl_ref[...], preferred_element_type=jnp.float32)
    s_ref[...] = (jnp.dot(xb, wr_ref[...], preferred_element_type=jnp.float32)
                  + b_ref[...])

  return pl.pallas_call(
      body,
      grid=(n // blk,),
      in_specs=[
          pl.BlockSpec((blk, d), lambda i: (i, 0)),
          pl.BlockSpec((d, f), lambda i: (0, 0)),
          pl.BlockSpec((d, f), lambda i: (0, 0)),
          pl.BlockSpec((1, f), lambda i: (0, 0)),
      ],
      out_specs=[
          pl.BlockSpec((blk, f), lambda i: (i, 0)),
          pl.BlockSpec((blk, f), lambda i: (i, 0)),
      ],
      out_shape=[
          jax.ShapeDtypeStruct((n, f), jnp.float32),
          jax.ShapeDtypeStruct((n, f), jnp.float32),
      ],
  )(x, w_l, w_r, b.reshape(1, f))


def _dense_mid(p1, dg, s1, w2_r, b2, blk):
  n, f = s1.shape
  n_pad = p1.shape[1]
  k = w2_r.shape[1]

  def body(p_ref, d_ref, s1_ref, w_ref, b_ref, h_ref, s2_ref):
    p = p_ref[0] + p_ref[1]
    deg = d_ref[0, :, 0:1] + d_ref[1, :, 0:1]
    h = jnp.maximum(p / jnp.maximum(deg, 1.0) + s1_ref[...], 0.0)
    h_ref[...] = h
    s2_ref[...] = (jnp.dot(h, w_ref[...], preferred_element_type=jnp.float32)
                   + b_ref[...])

  return pl.pallas_call(
      body,
      grid=(n // blk,),
      in_specs=[
          pl.BlockSpec((_NC, blk, _F), lambda i: (0, i, 0)),
          pl.BlockSpec((_NC, blk, _F), lambda i: (0, i, 0)),
          pl.BlockSpec((blk, f), lambda i: (i, 0)),
          pl.BlockSpec((f, k), lambda i: (0, 0)),
          pl.BlockSpec((1, k), lambda i: (0, 0)),
      ],
      out_specs=[
          pl.BlockSpec((blk, f), lambda i: (i, 0)),
          pl.BlockSpec((blk, k), lambda i: (i, 0)),
      ],
      out_shape=[
          jax.ShapeDtypeStruct((n, f), jnp.float32),
          jax.ShapeDtypeStruct((n, k), jnp.float32),
      ],
  )(p1, dg, s1, w2_r, b2.reshape(1, k))


def _dense_out(p2, dg, s2, w2_l, blk):
  n, k = s2.shape
  f = w2_l.shape[0]

  def body(p_ref, d_ref, s2_ref, w_ref, o_ref):
    p = p_ref[0] + p_ref[1]
    deg = d_ref[0, :, 0:1] + d_ref[1, :, 0:1]
    agg = p / jnp.maximum(deg, 1.0)
    lg = (jnp.dot(agg, w_ref[...], preferred_element_type=jnp.float32)
          + s2_ref[...])
    m = jnp.max(lg, axis=1, keepdims=True)
    lse = jnp.log(jnp.sum(jnp.exp(lg - m), axis=1, keepdims=True))
    o_ref[...] = lg - m - lse

  return pl.pallas_call(
      body,
      grid=(n // blk,),
      in_specs=[
          pl.BlockSpec((_NC, blk, _F), lambda i: (0, i, 0)),
          pl.BlockSpec((_NC, blk, _F), lambda i: (0, i, 0)),
          pl.BlockSpec((blk, k), lambda i: (i, 0)),
          pl.BlockSpec((f, k), lambda i: (0, 0)),
      ],
      out_specs=pl.BlockSpec((blk, k), lambda i: (i, 0)),
      out_shape=jax.ShapeDtypeStruct((n, k), jnp.float32),
  )(p2, dg, s2, w2_l)


def kernel(x, edge_index, W1_l, b1_l, W1_r, b1_r, W2_l, b2_l, W2_r, b2_r):
  n = x.shape[0]
  e = edge_index.shape[1]
  blk = 1000

  kpt = -(-e // (_NW * _CH))
  e_pad = _NW * _CH * kpt
  n_pad = -(-(n + 1) // (_NS * 8)) * (_NS * 8)

  src = edge_index[0].astype(jnp.int32)
  dst = edge_index[1].astype(jnp.int32)
  src_r = jnp.concatenate(
      [src, jnp.zeros((e_pad - e,), jnp.int32)]).reshape(_NW, kpt, _CH)
  dst_r = jnp.concatenate(
      [dst, jnp.full((e_pad - e,), n, jnp.int32)]).reshape(_NW, kpt, _CH)

  pad_rows = jnp.zeros((n_pad - n, _F), jnp.float32)
  y1, s1 = _dense_in(x, W1_l, W1_r, b1_l + b1_r, blk)
  p1, dg = _make_agg(n_pad, kpt, True)(
      src_r, dst_r, jnp.concatenate([y1, pad_rows]))
  h, s2 = _dense_mid(p1, dg, s1, W2_r, b2_l + b2_r, blk)
  (p2,) = _make_agg(n_pad, kpt, False)(
      src_r, dst_r, jnp.concatenate([h, pad_rows]))
  return _dense_out(p2, dg, s2, W2_l, blk)


# speedup vs baseline: 20.1781x; 1.0768x over previous
import functools

import jax
import jax.numpy as jnp
from jax import lax
from jax.experimental import pallas as pl
from jax.experimental.pallas import tpu as pltpu
from jax.experimental.pallas import tpu_sc as plsc

_NC = 2
_NS = 16
_NW = _NC * _NS
_CH = 128
_F = 16


def _make_agg(n_pad, kpt, with_deg):
  rows_pt = n_pad // _NS
  mesh = plsc.VectorSubcoreMesh(core_axis_name="c", subcore_axis_name="s")
  out_type = [jax.ShapeDtypeStruct((_NC, n_pad, _F), jnp.float32)]
  scratch = [
      pltpu.VMEM((kpt, _CH), jnp.int32),
      pltpu.VMEM((kpt, _CH), jnp.int32),
      pltpu.VMEM((_CH, _F), jnp.float32),
      pltpu.VMEM((_CH, _F), jnp.float32),
      pltpu.VMEM((rows_pt, _F), jnp.float32),
      pltpu.VMEM_SHARED((n_pad, _F), jnp.float32),
      pltpu.VMEM_SHARED((n_pad, _F), jnp.float32),
      pltpu.SemaphoreType.DMA,
      pltpu.SemaphoreType.DMA,
  ]
  if with_deg:
    out_type.append(jax.ShapeDtypeStruct((_NC, n_pad, _F), jnp.float32))
    scratch += [
        pltpu.VMEM((_CH, _F), jnp.float32),
        pltpu.VMEM_SHARED((n_pad, _F), jnp.float32),
        pltpu.SemaphoreType.DMA,
    ]

  def body(src_hbm, dst_hbm, tab_hbm, *refs):
    if with_deg:
      (out_hbm, deg_hbm, src_v, dst_v, rows0_v, rows1_v, tmp_v, acc_sh,
       tab_sh, sem0, sem1, ones_v, deg_sh, dsem) = refs
    else:
      (out_hbm, src_v, dst_v, rows0_v, rows1_v, tmp_v, acc_sh, tab_sh,
       sem0, sem1) = refs
    c = lax.axis_index("c")
    s = lax.axis_index("s")
    wid = s * _NC + c
    base = s * rows_pt

    pltpu.sync_copy(tab_hbm.at[pl.ds(base, rows_pt)], tmp_v)
    pltpu.sync_copy(tmp_v, tab_sh.at[pl.ds(base, rows_pt)])

    zero16 = jnp.zeros((_F,), jnp.float32)

    def zr(i, carry):
      tmp_v[i, :] = zero16
      return carry

    lax.fori_loop(0, rows_pt, zr, 0)
    pltpu.sync_copy(tmp_v, acc_sh.at[pl.ds(base, rows_pt)])
    if with_deg:
      pltpu.sync_copy(tmp_v, deg_sh.at[pl.ds(base, rows_pt)])
      one16 = jnp.ones((_F,), jnp.float32)

      def onr(i, carry):
        ones_v[i, :] = one16
        return carry

      lax.fori_loop(0, _CH, onr, 0)
    pltpu.sync_copy(src_hbm.at[wid], src_v)
    pltpu.sync_copy(dst_hbm.at[wid], dst_v)
    plsc.subcore_barrier()

    pltpu.async_copy(tab_sh.at[src_v.at[0]], rows0_v, sem0)

    def step(jj, carry):
      j0 = 2 * jj
      j1 = j0 + 1
      j2 = j0 + 2
      pltpu.make_async_copy(tab_sh.at[src_v.at[j0]], rows0_v, sem0).wait()
      pltpu.async_copy(tab_sh.at[src_v.at[j1]], rows1_v, sem1)
      pltpu.sync_copy(rows0_v, acc_sh.at[dst_v.at[j0]], add=True)
      if with_deg:
        pltpu.async_copy(ones_v, deg_sh.at[dst_v.at[j0]], dsem, add=True)
      pltpu.make_async_copy(tab_sh.at[src_v.at[j1]], rows1_v, sem1).wait()

      @pl.when(j2 < kpt)
      def _():
        pltpu.async_copy(tab_sh.at[src_v.at[j2]], rows0_v, sem0)

      pltpu.sync_copy(rows1_v, acc_sh.at[dst_v.at[j1]], add=True)
      if with_deg:
        pltpu.async_copy(ones_v, deg_sh.at[dst_v.at[j1]], dsem, add=True)
      return carry

    lax.fori_loop(0, kpt // 2, step, 0)
    if with_deg:

      def drain(j, carry):
        pltpu.make_async_copy(ones_v, deg_sh.at[dst_v.at[0]], dsem).wait()
        return carry

      lax.fori_loop(0, kpt, drain, 0)
    plsc.subcore_barrier()

    pltpu.sync_copy(acc_sh.at[pl.ds(base, rows_pt)], tmp_v)
    pltpu.sync_copy(tmp_v, out_hbm.at[c].at[pl.ds(base, rows_pt)])
    if with_deg:
      pltpu.sync_copy(deg_sh.at[pl.ds(base, rows_pt)], tmp_v)
      pltpu.sync_copy(tmp_v, deg_hbm.at[c].at[pl.ds(base, rows_pt)])

  return pl.kernel(
      body, out_type=out_type, mesh=mesh, scratch_types=scratch,
      compiler_params=pltpu.CompilerParams(use_tc_tiling_on_sc=False))


def _dense_in(x, w_l, w_r, b, blk):
  n, d = x.shape
  f = w_l.shape[1]

  def body(x_ref, wl_ref, wr_ref, b_ref, y_ref, s_ref):
    xb = x_ref[...]
    y_ref[...] = jnp.dot(xb, wl_ref[...], preferred_element_type=jnp.float32)
    s_ref[...] = (jnp.dot(xb, wr_ref[...], preferred_element_type=jnp.float32)
                  + b_ref[...])

  return pl.pallas_call(
      body,
      grid=(n // blk,),
      in_specs=[
          pl.BlockSpec((blk, d), lambda i: (i, 0)),
          pl.BlockSpec((d, f), lambda i: (0, 0)),
          pl.BlockSpec((d, f), lambda i: (0, 0)),
          pl.BlockSpec((1, f), lambda i: (0, 0)),
      ],
      out_specs=[
          pl.BlockSpec((blk, f), lambda i: (i, 0)),
          pl.BlockSpec((blk, f), lambda i: (i, 0)),
      ],
      out_shape=[
          jax.ShapeDtypeStruct((n, f), jnp.float32),
          jax.ShapeDtypeStruct((n, f), jnp.float32),
      ],
  )(x, w_l, w_r, b.reshape(1, f))


def _dense_mid(p1, dg, s1, w2_r, b2, blk):
  n, f = s1.shape
  n_pad = p1.shape[1]
  k = w2_r.shape[1]

  def body(p_ref, d_ref, s1_ref, w_ref, b_ref, h_ref, s2_ref):
    p = p_ref[0] + p_ref[1]
    deg = d_ref[0, :, 0:1] + d_ref[1, :, 0:1]
    h = jnp.maximum(p / jnp.maximum(deg, 1.0) + s1_ref[...], 0.0)
    h_ref[...] = h
    s2_ref[...] = (jnp.dot(h, w_ref[...], preferred_element_type=jnp.float32)
                   + b_ref[...])

  return pl.pallas_call(
      body,
      grid=(n // blk,),
      in_specs=[
          pl.BlockSpec((_NC, blk, _F), lambda i: (0, i, 0)),
          pl.BlockSpec((_NC, blk, _F), lambda i: (0, i, 0)),
          pl.BlockSpec((blk, f), lambda i: (i, 0)),
          pl.BlockSpec((f, k), lambda i: (0, 0)),
          pl.BlockSpec((1, k), lambda i: (0, 0)),
      ],
      out_specs=[
          pl.BlockSpec((blk, f), lambda i: (i, 0)),
          pl.BlockSpec((blk, k), lambda i: (i, 0)),
      ],
      out_shape=[
          jax.ShapeDtypeStruct((n, f), jnp.float32),
          jax.ShapeDtypeStruct((n, k), jnp.float32),
      ],
  )(p1, dg, s1, w2_r, b2.reshape(1, k))


def _dense_out(p2, dg, s2, w2_l, blk):
  n, k = s2.shape
  f = w2_l.shape[0]

  def body(p_ref, d_ref, s2_ref, w_ref, o_ref):
    p = p_ref[0] + p_ref[1]
    deg = d_ref[0, :, 0:1] + d_ref[1, :, 0:1]
    agg = p / jnp.maximum(deg, 1.0)
    lg = (jnp.dot(agg, w_ref[...], preferred_element_type=jnp.float32)
          + s2_ref[...])
    m = jnp.max(lg, axis=1, keepdims=True)
    lse = jnp.log(jnp.sum(jnp.exp(lg - m), axis=1, keepdims=True))
    o_ref[...] = lg - m - lse

  return pl.pallas_call(
      body,
      grid=(n // blk,),
      in_specs=[
          pl.BlockSpec((_NC, blk, _F), lambda i: (0, i, 0)),
          pl.BlockSpec((_NC, blk, _F), lambda i: (0, i, 0)),
          pl.BlockSpec((blk, k), lambda i: (i, 0)),
          pl.BlockSpec((f, k), lambda i: (0, 0)),
      ],
      out_specs=pl.BlockSpec((blk, k), lambda i: (i, 0)),
      out_shape=jax.ShapeDtypeStruct((n, k), jnp.float32),
  )(p2, dg, s2, w2_l)


def kernel(x, edge_index, W1_l, b1_l, W1_r, b1_r, W2_l, b2_l, W2_r, b2_r):
  n = x.shape[0]
  e = edge_index.shape[1]
  blk = 1000

  kpt = -(-e // (_NW * _CH))
  kpt += kpt % 2
  e_pad = _NW * _CH * kpt
  n_pad = -(-(n + 1) // (_NS * 8)) * (_NS * 8)

  src = edge_index[0].astype(jnp.int32)
  dst = edge_index[1].astype(jnp.int32)
  src_r = jnp.concatenate(
      [src, jnp.zeros((e_pad - e,), jnp.int32)]).reshape(_NW, kpt, _CH)
  dst_r = jnp.concatenate(
      [dst, jnp.full((e_pad - e,), n, jnp.int32)]).reshape(_NW, kpt, _CH)

  pad_rows = jnp.zeros((n_pad - n, _F), jnp.float32)
  y1, s1 = _dense_in(x, W1_l, W1_r, b1_l + b1_r, blk)
  p1, dg = _make_agg(n_pad, kpt, True)(
      src_r, dst_r, jnp.concatenate([y1, pad_rows]))
  h, s2 = _dense_mid(p1, dg, s1, W2_r, b2_l + b2_r, blk)
  (p2,) = _make_agg(n_pad, kpt, False)(
      src_r, dst_r, jnp.concatenate([h, pad_rows]))
  return _dense_out(p2, dg, s2, W2_l, blk)


# speedup vs baseline: 21.0986x; 1.0456x over previous
import functools

import jax
import jax.numpy as jnp
from jax import lax
from jax.experimental import pallas as pl
from jax.experimental.pallas import tpu as pltpu
from jax.experimental.pallas import tpu_sc as plsc

_NC = 2
_NS = 16
_NW = _NC * _NS
_CH = 128
_KB = 8
_CHB = _KB * _CH
_F = 16


def _make_agg(n, n_pad, kpt, with_deg):
  rows_pt = n_pad // _NS
  mesh = plsc.VectorSubcoreMesh(core_axis_name="c", subcore_axis_name="s")
  out_type = [jax.ShapeDtypeStruct((_NC, n_pad, _F), jnp.float32)]
  scratch = [
      pltpu.VMEM((kpt // _KB, _CHB), jnp.int32),
      pltpu.VMEM((kpt // _KB, _CHB), jnp.int32),
      pltpu.VMEM((_CHB, _F), jnp.float32),
      pltpu.VMEM((_CHB, _F), jnp.float32),
      pltpu.VMEM((rows_pt, _F), jnp.float32),
      pltpu.VMEM_SHARED((n_pad, _F), jnp.float32),
      pltpu.VMEM_SHARED((n_pad, _F), jnp.float32),
      pltpu.SemaphoreType.DMA,
      pltpu.SemaphoreType.DMA,
  ]
  if with_deg:
    out_type.append(jax.ShapeDtypeStruct((_NC, n_pad, _F), jnp.float32))
    scratch += [
        pltpu.VMEM((_CHB, _F), jnp.float32),
        pltpu.VMEM_SHARED((n_pad, _F), jnp.float32),
        pltpu.SemaphoreType.DMA,
    ]

  tail_pt = n - (_NS - 1) * rows_pt

  def body(eix_hbm, tab_hbm, *refs):
    if with_deg:
      (out_hbm, deg_hbm, src_v, dst_v, rows0_v, rows1_v, tmp_v, acc_sh,
       tab_sh, sem0, sem1, ones_v, deg_sh, dsem) = refs
    else:
      (out_hbm, src_v, dst_v, rows0_v, rows1_v, tmp_v, acc_sh, tab_sh,
       sem0, sem1) = refs
    c = lax.axis_index("c")
    s = lax.axis_index("s")
    wid = s * _NC + c
    base = s * rows_pt

    @pl.when(s < _NS - 1)
    def _():
      pltpu.sync_copy(tab_hbm.at[pl.ds(base, rows_pt)], tmp_v)
      pltpu.sync_copy(tmp_v, tab_sh.at[pl.ds(base, rows_pt)])

    @pl.when(s == _NS - 1)
    def _():
      pltpu.sync_copy(tab_hbm.at[pl.ds((_NS - 1) * rows_pt, tail_pt)],
                      tmp_v.at[pl.ds(0, tail_pt)])
      pltpu.sync_copy(tmp_v.at[pl.ds(0, tail_pt)],
                      tab_sh.at[pl.ds((_NS - 1) * rows_pt, tail_pt)])

    zero16 = jnp.zeros((_F,), jnp.float32)

    def zr(i, carry):
      tmp_v[i, :] = zero16
      return carry

    lax.fori_loop(0, rows_pt, zr, 0)
    pltpu.sync_copy(tmp_v, acc_sh.at[pl.ds(base, rows_pt)])
    if with_deg:
      pltpu.sync_copy(tmp_v, deg_sh.at[pl.ds(base, rows_pt)])
      one16 = jnp.ones((_F,), jnp.float32)

      def onr(i, carry):
        ones_v[i, :] = one16
        return carry

      lax.fori_loop(0, _CHB, onr, 0)
    pltpu.sync_copy(eix_hbm.at[0].at[wid], src_v)
    pltpu.sync_copy(eix_hbm.at[1].at[wid], dst_v)
    plsc.subcore_barrier()

    steps = kpt // _KB

    def gidx(j):
      return src_v.at[j]

    def sidx(j):
      return dst_v.at[j]

    pltpu.async_copy(tab_sh.at[gidx(0)], rows0_v, sem0)

    def step(jj, carry):
      j0 = 2 * jj
      j1 = j0 + 1
      j2 = j0 + 2
      pltpu.make_async_copy(tab_sh.at[gidx(j0)], rows0_v, sem0).wait()
      pltpu.async_copy(tab_sh.at[gidx(j1)], rows1_v, sem1)
      pltpu.sync_copy(rows0_v, acc_sh.at[sidx(j0)], add=True)
      if with_deg:
        pltpu.async_copy(ones_v, deg_sh.at[sidx(j0)], dsem, add=True)
      pltpu.make_async_copy(tab_sh.at[gidx(j1)], rows1_v, sem1).wait()

      @pl.when(j2 < steps)
      def _():
        pltpu.async_copy(tab_sh.at[gidx(j2)], rows0_v, sem0)

      pltpu.sync_copy(rows1_v, acc_sh.at[sidx(j1)], add=True)
      if with_deg:
        pltpu.async_copy(ones_v, deg_sh.at[sidx(j1)], dsem, add=True)
      return carry

    lax.fori_loop(0, steps // 2, step, 0)
    if with_deg:

      def drain(j, carry):
        pltpu.make_async_copy(ones_v, deg_sh.at[sidx(0)], dsem).wait()
        return carry

      lax.fori_loop(0, steps, drain, 0)
    plsc.subcore_barrier()

    pltpu.sync_copy(acc_sh.at[pl.ds(base, rows_pt)], tmp_v)
    pltpu.sync_copy(tmp_v, out_hbm.at[c].at[pl.ds(base, rows_pt)])
    if with_deg:
      pltpu.sync_copy(deg_sh.at[pl.ds(base, rows_pt)], tmp_v)
      pltpu.sync_copy(tmp_v, deg_hbm.at[c].at[pl.ds(base, rows_pt)])

  return pl.kernel(
      body, out_type=out_type, mesh=mesh, scratch_types=scratch,
      compiler_params=pltpu.CompilerParams(use_tc_tiling_on_sc=False))


def _dense_in(x, w_l, w_r, b, blk):
  n, d = x.shape
  f = w_l.shape[1]

  def body(x_ref, wl_ref, wr_ref, b_ref, y_ref, s_ref):
    xb = x_ref[...]
    y_ref[...] = jnp.dot(xb, wl_ref[...], preferred_element_type=jnp.float32)
    s_ref[...] = (jnp.dot(xb, wr_ref[...], preferred_element_type=jnp.float32)
                  + b_ref[...])

  return pl.pallas_call(
      body,
      grid=(n // blk,),
      in_specs=[
          pl.BlockSpec((blk, d), lambda i: (i, 0)),
          pl.BlockSpec((d, f), lambda i: (0, 0)),
          pl.BlockSpec((d, f), lambda i: (0, 0)),
          pl.BlockSpec((1, f), lambda i: (0, 0)),
      ],
      out_specs=[
          pl.BlockSpec((blk, f), lambda i: (i, 0)),
          pl.BlockSpec((blk, f), lambda i: (i, 0)),
      ],
      out_shape=[
          jax.ShapeDtypeStruct((n, f), jnp.float32),
          jax.ShapeDtypeStruct((n, f), jnp.float32),
      ],
  )(x, w_l, w_r, b.reshape(1, f))


def _dense_mid(p1, dg, s1, w2_r, b2, blk):
  n, f = s1.shape
  n_pad = p1.shape[1]
  k = w2_r.shape[1]

  def body(p_ref, d_ref, s1_ref, w_ref, b_ref, h_ref, s2_ref):
    p = p_ref[0] + p_ref[1]
    deg = d_ref[0, :, 0:1] + d_ref[1, :, 0:1]
    h = jnp.maximum(p / jnp.maximum(deg, 1.0) + s1_ref[...], 0.0)
    h_ref[...] = h
    s2_ref[...] = (jnp.dot(h, w_ref[...], preferred_element_type=jnp.float32)
                   + b_ref[...])

  return pl.pallas_call(
      body,
      grid=(n // blk,),
      in_specs=[
          pl.BlockSpec((_NC, blk, _F), lambda i: (0, i, 0)),
          pl.BlockSpec((_NC, blk, _F), lambda i: (0, i, 0)),
          pl.BlockSpec((blk, f), lambda i: (i, 0)),
          pl.BlockSpec((f, k), lambda i: (0, 0)),
          pl.BlockSpec((1, k), lambda i: (0, 0)),
      ],
      out_specs=[
          pl.BlockSpec((blk, f), lambda i: (i, 0)),
          pl.BlockSpec((blk, k), lambda i: (i, 0)),
      ],
      out_shape=[
          jax.ShapeDtypeStruct((n, f), jnp.float32),
          jax.ShapeDtypeStruct((n, k), jnp.float32),
      ],
  )(p1, dg, s1, w2_r, b2.reshape(1, k))


def _dense_out(p2, dg, s2, w2_l, blk):
  n, k = s2.shape
  f = w2_l.shape[0]

  def body(p_ref, d_ref, s2_ref, w_ref, o_ref):
    p = p_ref[0] + p_ref[1]
    deg = d_ref[0, :, 0:1] + d_ref[1, :, 0:1]
    agg = p / jnp.maximum(deg, 1.0)
    lg = (jnp.dot(agg, w_ref[...], preferred_element_type=jnp.float32)
          + s2_ref[...])
    m = jnp.max(lg, axis=1, keepdims=True)
    lse = jnp.log(jnp.sum(jnp.exp(lg - m), axis=1, keepdims=True))
    o_ref[...] = lg - m - lse

  return pl.pallas_call(
      body,
      grid=(n // blk,),
      in_specs=[
          pl.BlockSpec((_NC, blk, _F), lambda i: (0, i, 0)),
          pl.BlockSpec((_NC, blk, _F), lambda i: (0, i, 0)),
          pl.BlockSpec((blk, k), lambda i: (i, 0)),
          pl.BlockSpec((f, k), lambda i: (0, 0)),
      ],
      out_specs=pl.BlockSpec((blk, k), lambda i: (i, 0)),
      out_shape=jax.ShapeDtypeStruct((n, k), jnp.float32),
  )(p2, dg, s2, w2_l)


def kernel(x, edge_index, W1_l, b1_l, W1_r, b1_r, W2_l, b2_l, W2_r, b2_r):
  n = x.shape[0]
  e = edge_index.shape[1]
  blk = 1000

  kpt = -(-e // (_NW * _CH * 2 * _KB)) * 2 * _KB
  e_pad = _NW * _CH * kpt
  n_pad = -(-(n + 1) // (_NS * 8)) * (_NS * 8)

  pad_cols = jnp.concatenate([
      jnp.zeros((1, e_pad - e), jnp.int32),
      jnp.full((1, e_pad - e), n, jnp.int32)])
  eix = jnp.concatenate(
      [edge_index.astype(jnp.int32), pad_cols], axis=1
  ).reshape(2, _NW, kpt // _KB, _CHB)

  y1, s1 = _dense_in(x, W1_l, W1_r, b1_l + b1_r, blk)
  p1, dg = _make_agg(n, n_pad, kpt, True)(eix, y1)
  h, s2 = _dense_mid(p1, dg, s1, W2_r, b2_l + b2_r, blk)
  (p2,) = _make_agg(n, n_pad, kpt, False)(eix, h)
  return _dense_out(p2, dg, s2, W2_l, blk)


# speedup vs baseline: 21.9107x; 1.0385x over previous
import functools

import jax
import jax.numpy as jnp
from jax import lax
from jax.experimental import pallas as pl
from jax.experimental.pallas import tpu as pltpu
from jax.experimental.pallas import tpu_sc as plsc

_NC = 2
_NS = 16
_NW = _NC * _NS
_CH = 128
_KB = 8
_CHB = _KB * _CH
_F = 16


def _make_agg(n, n_pad, kpt, with_deg):
  rows_pt = n_pad // _NS
  mesh = plsc.VectorSubcoreMesh(core_axis_name="c", subcore_axis_name="s")
  out_type = [jax.ShapeDtypeStruct((_NC, n_pad, _F), jnp.float32)]
  scratch = [
      pltpu.VMEM((kpt // _KB, _CHB), jnp.int32),
      pltpu.VMEM((kpt // _KB, _CHB), jnp.int32),
      pltpu.VMEM((_CHB, _F), jnp.float32),
      pltpu.VMEM((_CHB, _F), jnp.float32),
      pltpu.VMEM((rows_pt, _F), jnp.float32),
      pltpu.VMEM_SHARED((n_pad, _F), jnp.float32),
      pltpu.VMEM_SHARED((n_pad, _F), jnp.float32),
      pltpu.SemaphoreType.DMA,
      pltpu.SemaphoreType.DMA,
  ]
  if with_deg:
    out_type.append(jax.ShapeDtypeStruct((_NC, n_pad, _F), jnp.float32))
    scratch += [
        pltpu.VMEM((_CHB, _F), jnp.float32),
        pltpu.VMEM_SHARED((n_pad, _F), jnp.float32),
        pltpu.SemaphoreType.DMA,
    ]

  tail_pt = n - (_NS - 1) * rows_pt

  def body(eix_hbm, tab_hbm, *refs):
    if with_deg:
      (out_hbm, deg_hbm, src_v, dst_v, rows0_v, rows1_v, tmp_v, acc_sh,
       tab_sh, sem0, sem1, ones_v, deg_sh, dsem) = refs
    else:
      (out_hbm, src_v, dst_v, rows0_v, rows1_v, tmp_v, acc_sh, tab_sh,
       sem0, sem1) = refs
    c = lax.axis_index("c")
    s = lax.axis_index("s")
    wid = s * _NC + c
    base = s * rows_pt

    @pl.when(s < _NS - 1)
    def _():
      pltpu.sync_copy(tab_hbm.at[pl.ds(base, rows_pt)], tmp_v)
      pltpu.sync_copy(tmp_v, tab_sh.at[pl.ds(base, rows_pt)])

    @pl.when(s == _NS - 1)
    def _():
      pltpu.sync_copy(tab_hbm.at[pl.ds((_NS - 1) * rows_pt, tail_pt)],
                      tmp_v.at[pl.ds(0, tail_pt)])
      pltpu.sync_copy(tmp_v.at[pl.ds(0, tail_pt)],
                      tab_sh.at[pl.ds((_NS - 1) * rows_pt, tail_pt)])

    zero16 = jnp.zeros((_F,), jnp.float32)

    def zr(i, carry):
      tmp_v[i, :] = zero16
      return carry

    lax.fori_loop(0, rows_pt, zr, 0)
    pltpu.sync_copy(tmp_v, acc_sh.at[pl.ds(base, rows_pt)])
    if with_deg:
      pltpu.sync_copy(tmp_v, deg_sh.at[pl.ds(base, rows_pt)])
      one16 = jnp.ones((_F,), jnp.float32)

      def onr(i, carry):
        ones_v[i, :] = one16
        return carry

      lax.fori_loop(0, _CHB, onr, 0)
    pltpu.sync_copy(eix_hbm.at[0].at[wid], src_v)
    pltpu.sync_copy(eix_hbm.at[1].at[wid], dst_v)
    plsc.subcore_barrier()

    steps = kpt // _KB

    def gidx(j):
      return src_v.at[j]

    def sidx(j):
      return dst_v.at[j]

    pltpu.async_copy(tab_sh.at[gidx(0)], rows0_v, sem0)

    def step(jj, carry):
      j0 = 2 * jj
      j1 = j0 + 1
      j2 = j0 + 2
      pltpu.make_async_copy(tab_sh.at[gidx(j0)], rows0_v, sem0).wait()
      pltpu.async_copy(tab_sh.at[gidx(j1)], rows1_v, sem1)
      pltpu.sync_copy(rows0_v, acc_sh.at[sidx(j0)], add=True)
      if with_deg:
        pltpu.async_copy(ones_v, deg_sh.at[sidx(j0)], dsem, add=True)
      pltpu.make_async_copy(tab_sh.at[gidx(j1)], rows1_v, sem1).wait()

      @pl.when(j2 < steps)
      def _():
        pltpu.async_copy(tab_sh.at[gidx(j2)], rows0_v, sem0)

      pltpu.sync_copy(rows1_v, acc_sh.at[sidx(j1)], add=True)
      if with_deg:
        pltpu.async_copy(ones_v, deg_sh.at[sidx(j1)], dsem, add=True)
      return carry

    lax.fori_loop(0, steps // 2, step, 0)
    if with_deg:

      def drain(j, carry):
        pltpu.make_async_copy(ones_v, deg_sh.at[sidx(0)], dsem).wait()
        return carry

      lax.fori_loop(0, steps, drain, 0)
    plsc.subcore_barrier()

    pltpu.sync_copy(acc_sh.at[pl.ds(base, rows_pt)], tmp_v)
    pltpu.sync_copy(tmp_v, out_hbm.at[c].at[pl.ds(base, rows_pt)])
    if with_deg:
      pltpu.sync_copy(deg_sh.at[pl.ds(base, rows_pt)], tmp_v)
      pltpu.sync_copy(tmp_v, deg_hbm.at[c].at[pl.ds(base, rows_pt)])

  return pl.kernel(
      body, out_type=out_type, mesh=mesh, scratch_types=scratch,
      compiler_params=pltpu.CompilerParams(use_tc_tiling_on_sc=False))


def _make_agg2h(n, n_pad, kpt):
  rows_pt = n_pad // _NS
  mesh = plsc.VectorSubcoreMesh(core_axis_name="c", subcore_axis_name="s")
  out_type = [jax.ShapeDtypeStruct((_NC, n_pad, _F), jnp.float32)]
  scratch = [
      pltpu.VMEM((kpt // _KB, _CHB), jnp.int32),
      pltpu.VMEM((kpt // _KB, _CHB), jnp.int32),
      pltpu.VMEM((_CHB, _F), jnp.float32),
      pltpu.VMEM((_CHB, _F), jnp.float32),
      pltpu.VMEM((rows_pt, _F), jnp.float32),
      pltpu.VMEM((rows_pt, _F), jnp.float32),
      pltpu.VMEM((rows_pt, _F), jnp.float32),
      pltpu.VMEM_SHARED((n_pad, _F), jnp.float32),
      pltpu.VMEM_SHARED((n_pad, _F), jnp.float32),
      pltpu.SemaphoreType.DMA,
      pltpu.SemaphoreType.DMA,
  ]

  tail_pt = n - (_NS - 1) * rows_pt

  def body(eix_hbm, p1_hbm, dg_hbm, s1_hbm, out_hbm, src_v, dst_v,
           rows0_v, rows1_v, tmp_v, a_v, b_v, acc_sh, tab_sh, sem0, sem1):
    c = lax.axis_index("c")
    s = lax.axis_index("s")
    wid = s * _NC + c
    base = s * rows_pt
    my_pt = jnp.where(s == _NS - 1, tail_pt, rows_pt)

    pltpu.sync_copy(p1_hbm.at[0].at[pl.ds(base, rows_pt)], a_v)
    pltpu.sync_copy(p1_hbm.at[1].at[pl.ds(base, rows_pt)], b_v)

    def addf(i, carry):
      a_v[i, :] = a_v[i, :] + b_v[i, :]
      return carry

    lax.fori_loop(0, my_pt, addf, 0)
    pltpu.sync_copy(dg_hbm.at[0].at[pl.ds(base, rows_pt)], tmp_v)
    pltpu.sync_copy(dg_hbm.at[1].at[pl.ds(base, rows_pt)], b_v)

    def degf(i, carry):
      b_v[i, :] = a_v[i, :] / jnp.maximum(tmp_v[i, :] + b_v[i, :], 1.0)
      return carry

    lax.fori_loop(0, my_pt, degf, 0)

    @pl.when(s < _NS - 1)
    def _():
      pltpu.sync_copy(s1_hbm.at[pl.ds(base, rows_pt)], a_v)

    @pl.when(s == _NS - 1)
    def _():
      pltpu.sync_copy(s1_hbm.at[pl.ds((_NS - 1) * rows_pt, tail_pt)],
                      a_v.at[pl.ds(0, tail_pt)])

    zero16 = jnp.zeros((_F,), jnp.float32)

    def huf(i, carry):
      a_v[i, :] = jnp.maximum(a_v[i, :] + b_v[i, :], zero16)
      return carry

    lax.fori_loop(0, my_pt, huf, 0)

    @pl.when(s < _NS - 1)
    def _():
      pltpu.sync_copy(a_v, tab_sh.at[pl.ds(base, rows_pt)])

    @pl.when(s == _NS - 1)
    def _():
      pltpu.sync_copy(a_v.at[pl.ds(0, tail_pt)],
                      tab_sh.at[pl.ds((_NS - 1) * rows_pt, tail_pt)])

    def zr(i, carry):
      tmp_v[i, :] = zero16
      return carry

    lax.fori_loop(0, rows_pt, zr, 0)
    pltpu.sync_copy(tmp_v, acc_sh.at[pl.ds(base, rows_pt)])
    pltpu.sync_copy(eix_hbm.at[0].at[wid], src_v)
    pltpu.sync_copy(eix_hbm.at[1].at[wid], dst_v)
    plsc.subcore_barrier()

    steps = kpt // _KB
    pltpu.async_copy(tab_sh.at[src_v.at[0]], rows0_v, sem0)

    def step(jj, carry):
      j0 = 2 * jj
      j1 = j0 + 1
      j2 = j0 + 2
      pltpu.make_async_copy(tab_sh.at[src_v.at[j0]], rows0_v, sem0).wait()
      pltpu.async_copy(tab_sh.at[src_v.at[j1]], rows1_v, sem1)
      pltpu.sync_copy(rows0_v, acc_sh.at[dst_v.at[j0]], add=True)
      pltpu.make_async_copy(tab_sh.at[src_v.at[j1]], rows1_v, sem1).wait()

      @pl.when(j2 < steps)
      def _():
        pltpu.async_copy(tab_sh.at[src_v.at[j2]], rows0_v, sem0)

      pltpu.sync_copy(rows1_v, acc_sh.at[dst_v.at[j1]], add=True)
      return carry

    lax.fori_loop(0, steps // 2, step, 0)
    plsc.subcore_barrier()

    pltpu.sync_copy(acc_sh.at[pl.ds(base, rows_pt)], tmp_v)
    pltpu.sync_copy(tmp_v, out_hbm.at[c].at[pl.ds(base, rows_pt)])

  return pl.kernel(
      body, out_type=out_type, mesh=mesh, scratch_types=scratch,
      compiler_params=pltpu.CompilerParams(use_tc_tiling_on_sc=False))


def _dense_in(x, w_l, w_r, b, blk):
  n, d = x.shape
  f = w_l.shape[1]

  def body(x_ref, wl_ref, wr_ref, b_ref, y_ref, s_ref):
    xb = x_ref[...]
    y_ref[...] = jnp.dot(xb, wl_ref[...], preferred_element_type=jnp.float32)
    s_ref[...] = (jnp.dot(xb, wr_ref[...], preferred_element_type=jnp.float32)
                  + b_ref[...])

  return pl.pallas_call(
      body,
      grid=(n // blk,),
      in_specs=[
          pl.BlockSpec((blk, d), lambda i: (i, 0)),
          pl.BlockSpec((d, f), lambda i: (0, 0)),
          pl.BlockSpec((d, f), lambda i: (0, 0)),
          pl.BlockSpec((1, f), lambda i: (0, 0)),
      ],
      out_specs=[
          pl.BlockSpec((blk, f), lambda i: (i, 0)),
          pl.BlockSpec((blk, f), lambda i: (i, 0)),
      ],
      out_shape=[
          jax.ShapeDtypeStruct((n, f), jnp.float32),
          jax.ShapeDtypeStruct((n, f), jnp.float32),
      ],
  )(x, w_l, w_r, b.reshape(1, f))


def _dense_out(p1, dg, p2, s1, w2_l, w2_r, b2, blk):
  n, f = s1.shape
  k = w2_l.shape[1]

  def body(p1_ref, d_ref, p2_ref, s1_ref, wl_ref, wr_ref, b_ref, o_ref):
    deg = jnp.maximum(d_ref[0, :, 0:1] + d_ref[1, :, 0:1], 1.0)
    h = jnp.maximum((p1_ref[0] + p1_ref[1]) / deg + s1_ref[...], 0.0)
    agg = (p2_ref[0] + p2_ref[1]) / deg
    lg = (jnp.dot(agg, wl_ref[...], preferred_element_type=jnp.float32)
          + jnp.dot(h, wr_ref[...], preferred_element_type=jnp.float32)
          + b_ref[...])
    m = jnp.max(lg, axis=1, keepdims=True)
    lse = jnp.log(jnp.sum(jnp.exp(lg - m), axis=1, keepdims=True))
    o_ref[...] = lg - m - lse

  return pl.pallas_call(
      body,
      grid=(n // blk,),
      in_specs=[
          pl.BlockSpec((_NC, blk, _F), lambda i: (0, i, 0)),
          pl.BlockSpec((_NC, blk, _F), lambda i: (0, i, 0)),
          pl.BlockSpec((_NC, blk, _F), lambda i: (0, i, 0)),
          pl.BlockSpec((blk, f), lambda i: (i, 0)),
          pl.BlockSpec((f, k), lambda i: (0, 0)),
          pl.BlockSpec((f, k), lambda i: (0, 0)),
          pl.BlockSpec((1, k), lambda i: (0, 0)),
      ],
      out_specs=pl.BlockSpec((blk, k), lambda i: (i, 0)),
      out_shape=jax.ShapeDtypeStruct((n, k), jnp.float32),
  )(p1, dg, p2, s1, w2_l, w2_r, b2.reshape(1, k))


def kernel(x, edge_index, W1_l, b1_l, W1_r, b1_r, W2_l, b2_l, W2_r, b2_r):
  n = x.shape[0]
  e = edge_index.shape[1]
  blk = 1000

  kpt = -(-e // (_NW * _CH * 2 * _KB)) * 2 * _KB
  e_pad = _NW * _CH * kpt
  n_pad = -(-(n + 1) // (_NS * 8)) * (_NS * 8)

  pad_cols = jnp.concatenate([
      jnp.zeros((1, e_pad - e), jnp.int32),
      jnp.full((1, e_pad - e), n, jnp.int32)])
  eix = jnp.concatenate(
      [edge_index.astype(jnp.int32), pad_cols], axis=1
  ).reshape(2, _NW, kpt // _KB, _CHB)

  y1, s1 = _dense_in(x, W1_l, W1_r, b1_l + b1_r, blk)
  p1, dg = _make_agg(n, n_pad, kpt, True)(eix, y1)
  (p2,) = _make_agg2h(n, n_pad, kpt)(eix, p1, dg, s1)
  return _dense_out(p1, dg, p2, s1, W2_l, W2_r, b2_l + b2_r, blk)


# speedup vs baseline: 23.0885x; 1.0538x over previous
import functools

import jax
import jax.numpy as jnp
from jax import lax
from jax.experimental import pallas as pl
from jax.experimental.pallas import tpu as pltpu
from jax.experimental.pallas import tpu_sc as plsc

_NC = 2
_NS = 16
_NW = _NC * _NS
_CH = 128
_KB = 8
_CHB = _KB * _CH
_F = 16


def _make_agg(n, n_pad, kpt, with_deg):
  rows_pt = n_pad // _NS
  mesh = plsc.VectorSubcoreMesh(core_axis_name="c", subcore_axis_name="s")
  out_type = [jax.ShapeDtypeStruct((_NC, n_pad, _F), jnp.float32)]
  scratch = [
      pltpu.VMEM((kpt // _KB, _CHB), jnp.int32),
      pltpu.VMEM((kpt // _KB, _CHB), jnp.int32),
      pltpu.VMEM((_CHB, _F), jnp.float32),
      pltpu.VMEM((_CHB, _F), jnp.float32),
      pltpu.VMEM((rows_pt, _F), jnp.float32),
      pltpu.VMEM_SHARED((n_pad, _F), jnp.float32),
      pltpu.VMEM_SHARED((n_pad, _F), jnp.float32),
      pltpu.SemaphoreType.DMA,
      pltpu.SemaphoreType.DMA,
  ]
  if with_deg:
    out_type.append(jax.ShapeDtypeStruct((_NC, n_pad, _F), jnp.float32))
    scratch += [
        pltpu.VMEM((_CHB, _F), jnp.float32),
        pltpu.VMEM_SHARED((n_pad, _F), jnp.float32),
        pltpu.SemaphoreType.DMA,
    ]

  tail_pt = n - (_NS - 1) * rows_pt

  def body(eix_hbm, tab_hbm, *refs):
    if with_deg:
      (out_hbm, deg_hbm, src_v, dst_v, rows0_v, rows1_v, tmp_v, acc_sh,
       tab_sh, sem0, sem1, ones_v, deg_sh, dsem) = refs
    else:
      (out_hbm, src_v, dst_v, rows0_v, rows1_v, tmp_v, acc_sh, tab_sh,
       sem0, sem1) = refs
    c = lax.axis_index("c")
    s = lax.axis_index("s")
    wid = s * _NC + c
    base = s * rows_pt

    @pl.when(s < _NS - 1)
    def _():
      pltpu.sync_copy(tab_hbm.at[pl.ds(base, rows_pt)], tmp_v)
      pltpu.sync_copy(tmp_v, tab_sh.at[pl.ds(base, rows_pt)])

    @pl.when(s == _NS - 1)
    def _():
      pltpu.sync_copy(tab_hbm.at[pl.ds((_NS - 1) * rows_pt, tail_pt)],
                      tmp_v.at[pl.ds(0, tail_pt)])
      pltpu.sync_copy(tmp_v.at[pl.ds(0, tail_pt)],
                      tab_sh.at[pl.ds((_NS - 1) * rows_pt, tail_pt)])

    zero16 = jnp.zeros((_F,), jnp.float32)

    def zr(i, carry):
      tmp_v[i, :] = zero16
      return carry

    lax.fori_loop(0, rows_pt, zr, 0)
    pltpu.sync_copy(tmp_v, acc_sh.at[pl.ds(base, rows_pt)])
    if with_deg:
      pltpu.sync_copy(tmp_v, deg_sh.at[pl.ds(base, rows_pt)])
      one16 = jnp.ones((_F,), jnp.float32)

      def onr(i, carry):
        ones_v[i, :] = one16
        return carry

      lax.fori_loop(0, _CHB, onr, 0)
    pltpu.sync_copy(eix_hbm.at[0].at[wid], src_v)
    pltpu.sync_copy(eix_hbm.at[1].at[wid], dst_v)
    plsc.subcore_barrier()

    steps = kpt // _KB

    def gidx(j):
      return src_v.at[j]

    def sidx(j):
      return dst_v.at[j]

    pltpu.async_copy(tab_sh.at[gidx(0)], rows0_v, sem0)

    def step(jj, carry):
      j0 = 2 * jj
      j1 = j0 + 1
      j2 = j0 + 2
      pltpu.make_async_copy(tab_sh.at[gidx(j0)], rows0_v, sem0).wait()
      pltpu.async_copy(tab_sh.at[gidx(j1)], rows1_v, sem1)
      pltpu.sync_copy(rows0_v, acc_sh.at[sidx(j0)], add=True)
      if with_deg:
        pltpu.async_copy(ones_v, deg_sh.at[sidx(j0)], dsem, add=True)
      pltpu.make_async_copy(tab_sh.at[gidx(j1)], rows1_v, sem1).wait()

      @pl.when(j2 < steps)
      def _():
        pltpu.async_copy(tab_sh.at[gidx(j2)], rows0_v, sem0)

      pltpu.sync_copy(rows1_v, acc_sh.at[sidx(j1)], add=True)
      if with_deg:
        pltpu.async_copy(ones_v, deg_sh.at[sidx(j1)], dsem, add=True)
      return carry

    lax.fori_loop(0, steps // 2, step, 0)
    if with_deg:

      def drain(j, carry):
        pltpu.make_async_copy(ones_v, deg_sh.at[sidx(0)], dsem).wait()
        return carry

      lax.fori_loop(0, steps, drain, 0)
    plsc.subcore_barrier()

    pltpu.sync_copy(acc_sh.at[pl.ds(base, rows_pt)], tmp_v)
    pltpu.sync_copy(tmp_v, out_hbm.at[c].at[pl.ds(base, rows_pt)])
    if with_deg:
      pltpu.sync_copy(deg_sh.at[pl.ds(base, rows_pt)], tmp_v)
      pltpu.sync_copy(tmp_v, deg_hbm.at[c].at[pl.ds(base, rows_pt)])

  return pl.kernel(
      body, out_type=out_type, mesh=mesh, scratch_types=scratch,
      compiler_params=pltpu.CompilerParams(use_tc_tiling_on_sc=False))


def _make_agg2h(n, n_pad, kpt):
  rows_pt = n_pad // _NS
  mesh = plsc.VectorSubcoreMesh(core_axis_name="c", subcore_axis_name="s")
  out_type = [jax.ShapeDtypeStruct((_NC, n_pad, _F), jnp.float32)]
  scratch = [
      pltpu.VMEM((kpt // _KB, _CHB), jnp.int32),
      pltpu.VMEM((kpt // _KB, _CHB), jnp.int32),
      pltpu.VMEM((_CHB, _F), jnp.float32),
      pltpu.VMEM((_CHB, _F), jnp.float32),
      pltpu.VMEM((rows_pt, _F), jnp.float32),
      pltpu.VMEM((rows_pt, _F), jnp.float32),
      pltpu.VMEM((rows_pt, _F), jnp.float32),
      pltpu.VMEM((rows_pt, _F), jnp.float32),
      pltpu.VMEM((rows_pt, _F), jnp.float32),
      pltpu.VMEM_SHARED((n_pad, _F), jnp.float32),
      pltpu.VMEM_SHARED((n_pad, _F), jnp.float32),
      pltpu.SemaphoreType.DMA,
      pltpu.SemaphoreType.DMA,
  ]

  tail_pt = n - (_NS - 1) * rows_pt

  def body(eix_hbm, p1_hbm, dg_hbm, s1_hbm, out_hbm, src_v, dst_v,
           rows0_v, rows1_v, a_v, pa_v, pb_v, da_v, db_v,
           acc_sh, tab_sh, sem0, sem1):
    c = lax.axis_index("c")
    s = lax.axis_index("s")
    wid = s * _NC + c
    base = s * rows_pt
    my_pt = jnp.where(s == _NS - 1, tail_pt, rows_pt)

    pltpu.async_copy(p1_hbm.at[0].at[pl.ds(base, rows_pt)], pa_v, sem0)
    pltpu.async_copy(p1_hbm.at[1].at[pl.ds(base, rows_pt)], pb_v, sem0)
    pltpu.async_copy(dg_hbm.at[0].at[pl.ds(base, rows_pt)], da_v, sem0)
    pltpu.async_copy(dg_hbm.at[1].at[pl.ds(base, rows_pt)], db_v, sem0)

    @pl.when(s < _NS - 1)
    def _():
      pltpu.async_copy(s1_hbm.at[pl.ds(base, rows_pt)], a_v, sem1)

    @pl.when(s == _NS - 1)
    def _():
      pltpu.async_copy(s1_hbm.at[pl.ds((_NS - 1) * rows_pt, tail_pt)],
                       a_v.at[pl.ds(0, tail_pt)], sem1)

    pltpu.make_async_copy(p1_hbm.at[0].at[pl.ds(base, rows_pt)], pa_v,
                          sem0).wait()
    pltpu.make_async_copy(p1_hbm.at[1].at[pl.ds(base, rows_pt)], pb_v,
                          sem0).wait()
    pltpu.make_async_copy(dg_hbm.at[0].at[pl.ds(base, rows_pt)], da_v,
                          sem0).wait()
    pltpu.make_async_copy(dg_hbm.at[1].at[pl.ds(base, rows_pt)], db_v,
                          sem0).wait()

    @pl.when(s < _NS - 1)
    def _():
      pltpu.make_async_copy(s1_hbm.at[pl.ds(base, rows_pt)], a_v,
                            sem1).wait()

    @pl.when(s == _NS - 1)
    def _():
      pltpu.make_async_copy(s1_hbm.at[pl.ds((_NS - 1) * rows_pt, tail_pt)],
                            a_v.at[pl.ds(0, tail_pt)], sem1).wait()

    zero16 = jnp.zeros((_F,), jnp.float32)

    def huf(i, carry):
      dsum = jnp.maximum(da_v[i, :] + db_v[i, :], 1.0)
      a_v[i, :] = jnp.maximum(
          a_v[i, :] + (pa_v[i, :] + pb_v[i, :]) / dsum, zero16)
      return carry

    lax.fori_loop(0, my_pt, huf, 0)

    @pl.when(s < _NS - 1)
    def _():
      pltpu.sync_copy(a_v, tab_sh.at[pl.ds(base, rows_pt)])

    @pl.when(s == _NS - 1)
    def _():
      pltpu.sync_copy(a_v.at[pl.ds(0, tail_pt)],
                      tab_sh.at[pl.ds((_NS - 1) * rows_pt, tail_pt)])

    def zr(i, carry):
      pa_v[i, :] = zero16
      return carry

    lax.fori_loop(0, rows_pt, zr, 0)
    pltpu.sync_copy(pa_v, acc_sh.at[pl.ds(base, rows_pt)])
    pltpu.sync_copy(eix_hbm.at[0].at[wid], src_v)
    pltpu.sync_copy(eix_hbm.at[1].at[wid], dst_v)
    plsc.subcore_barrier()

    steps = kpt // _KB
    pltpu.async_copy(tab_sh.at[src_v.at[0]], rows0_v, sem0)

    def step(jj, carry):
      j0 = 2 * jj
      j1 = j0 + 1
      j2 = j0 + 2
      pltpu.make_async_copy(tab_sh.at[src_v.at[j0]], rows0_v, sem0).wait()
      pltpu.async_copy(tab_sh.at[src_v.at[j1]], rows1_v, sem1)
      pltpu.sync_copy(rows0_v, acc_sh.at[dst_v.at[j0]], add=True)
      pltpu.make_async_copy(tab_sh.at[src_v.at[j1]], rows1_v, sem1).wait()

      @pl.when(j2 < steps)
      def _():
        pltpu.async_copy(tab_sh.at[src_v.at[j2]], rows0_v, sem0)

      pltpu.sync_copy(rows1_v, acc_sh.at[dst_v.at[j1]], add=True)
      return carry

    lax.fori_loop(0, steps // 2, step, 0)
    plsc.subcore_barrier()

    pltpu.sync_copy(acc_sh.at[pl.ds(base, rows_pt)], pa_v)
    pltpu.sync_copy(pa_v, out_hbm.at[c].at[pl.ds(base, rows_pt)])

  return pl.kernel(
      body, out_type=out_type, mesh=mesh, scratch_types=scratch,
      compiler_params=pltpu.CompilerParams(use_tc_tiling_on_sc=False))


def _dense_in(x, w_l, w_r, b, blk):
  n, d = x.shape
  f = w_l.shape[1]

  def body(x_ref, wl_ref, wr_ref, b_ref, y_ref, s_ref):
    xb = x_ref[...]
    y_ref[...] = jnp.dot(xb, wl_ref[...], preferred_element_type=jnp.float32)
    s_ref[...] = (jnp.dot(xb, wr_ref[...], preferred_element_type=jnp.float32)
                  + b_ref[...])

  return pl.pallas_call(
      body,
      grid=(n // blk,),
      in_specs=[
          pl.BlockSpec((blk, d), lambda i: (i, 0)),
          pl.BlockSpec((d, f), lambda i: (0, 0)),
          pl.BlockSpec((d, f), lambda i: (0, 0)),
          pl.BlockSpec((1, f), lambda i: (0, 0)),
      ],
      out_specs=[
          pl.BlockSpec((blk, f), lambda i: (i, 0)),
          pl.BlockSpec((blk, f), lambda i: (i, 0)),
      ],
      out_shape=[
          jax.ShapeDtypeStruct((n, f), jnp.float32),
          jax.ShapeDtypeStruct((n, f), jnp.float32),
      ],
  )(x, w_l, w_r, b.reshape(1, f))


def _dense_out(p1, dg, p2, s1, w2_l, w2_r, b2, blk):
  n, f = s1.shape
  k = w2_l.shape[1]

  def body(p1_ref, d_ref, p2_ref, s1_ref, wl_ref, wr_ref, b_ref, o_ref):
    deg = jnp.maximum(d_ref[0, :, 0:1] + d_ref[1, :, 0:1], 1.0)
    h = jnp.maximum((p1_ref[0] + p1_ref[1]) / deg + s1_ref[...], 0.0)
    agg = (p2_ref[0] + p2_ref[1]) / deg
    lg = (jnp.dot(agg, wl_ref[...], preferred_element_type=jnp.float32)
          + jnp.dot(h, wr_ref[...], preferred_element_type=jnp.float32)
          + b_ref[...])
    m = jnp.max(lg, axis=1, keepdims=True)
    lse = jnp.log(jnp.sum(jnp.exp(lg - m), axis=1, keepdims=True))
    o_ref[...] = lg - m - lse

  return pl.pallas_call(
      body,
      grid=(n // blk,),
      in_specs=[
          pl.BlockSpec((_NC, blk, _F), lambda i: (0, i, 0)),
          pl.BlockSpec((_NC, blk, _F), lambda i: (0, i, 0)),
          pl.BlockSpec((_NC, blk, _F), lambda i: (0, i, 0)),
          pl.BlockSpec((blk, f), lambda i: (i, 0)),
          pl.BlockSpec((f, k), lambda i: (0, 0)),
          pl.BlockSpec((f, k), lambda i: (0, 0)),
          pl.BlockSpec((1, k), lambda i: (0, 0)),
      ],
      out_specs=pl.BlockSpec((blk, k), lambda i: (i, 0)),
      out_shape=jax.ShapeDtypeStruct((n, k), jnp.float32),
  )(p1, dg, p2, s1, w2_l, w2_r, b2.reshape(1, k))


def kernel(x, edge_index, W1_l, b1_l, W1_r, b1_r, W2_l, b2_l, W2_r, b2_r):
  n = x.shape[0]
  e = edge_index.shape[1]
  blk = 1000

  kpt = -(-e // (_NW * _CH * 2 * _KB)) * 2 * _KB
  e_pad = _NW * _CH * kpt
  n_pad = -(-(n + 1) // (_NS * 8)) * (_NS * 8)

  pad_cols = jnp.concatenate([
      jnp.zeros((1, e_pad - e), jnp.int32),
      jnp.full((1, e_pad - e), n, jnp.int32)])
  eix = jnp.concatenate(
      [edge_index.astype(jnp.int32), pad_cols], axis=1
  ).reshape(2, _NW, kpt // _KB, _CHB)

  y1, s1 = _dense_in(x, W1_l, W1_r, b1_l + b1_r, blk)
  p1, dg = _make_agg(n, n_pad, kpt, True)(eix, y1)
  (p2,) = _make_agg2h(n, n_pad, kpt)(eix, p1, dg, s1)
  return _dense_out(p1, dg, p2, s1, W2_l, W2_r, b2_l + b2_r, blk)
